# baseline (speedup 1.0000x reference)
import functools

import jax
import jax.numpy as jnp
from jax import lax
from jax.experimental import pallas as pl
from jax.experimental.pallas import tpu as pltpu
from jax.experimental.pallas import tpu_sc as plsc

PAD_IDX = 0
NEG_INF = -1e9

NC = 2
NS = 16
NW = NC * NS

GROUPS_PER_STEP = 8
CHUNK = GROUPS_PER_STEP * 128


def _sc_gather_kernel(n_rows, n_steps, bias_groups, do_bias,
                      table, idx_hbm, uidx, iidx, ubias, ibias,
                      out_hbm, ub_out, ib_out,
                      idx_v, rows_v0, rows_v1, bias_v, sem0, sem1):
    wid = lax.axis_index("s") * NC + lax.axis_index("c")
    rows_per_w = n_rows // NW
    idxrows_per_w = rows_per_w // 128
    bufs = (rows_v0, rows_v1)
    sems = (sem0, sem1)

    def fire(k, p):
        for j in range(GROUPS_PER_STEP):
            pltpu.async_copy(table.at[idx_v.at[k * GROUPS_PER_STEP + j]],
                             bufs[p].at[pl.ds(j * 128, 128)], sems[p])

    def drain(p):
        for j in range(GROUPS_PER_STEP):
            pltpu.make_async_copy(table.at[idx_v.at[j]],
                                  bufs[p].at[pl.ds(j * 128, 128)],
                                  sems[p]).wait()

    pltpu.sync_copy(
        idx_hbm.at[pl.ds(wid * idxrows_per_w, idxrows_per_w)], idx_v)
    fire(0, 0)

    def step2(m, carry):
        for j in range(2):
            k = 2 * m + j
            drain(j)
            fire(k + 1, 1 - j)
            pltpu.sync_copy(
                bufs[j],
                out_hbm.at[pl.ds(wid * rows_per_w + k * CHUNK, CHUNK)])
        return carry

    lax.fori_loop(0, (n_steps - 1) // 2, step2, 0)
    drain((n_steps - 1) % 2)
    pltpu.sync_copy(
        bufs[(n_steps - 1) % 2],
        out_hbm.at[pl.ds(wid * rows_per_w + (n_steps - 1) * CHUNK, CHUNK)])

    if do_bias:
        half = NW // 2
        for active, bidx_hbm, btab, bout in ((wid < half, uidx, ubias, ub_out),
                                             (wid >= half, iidx, ibias,
                                              ib_out)):
            @pl.when(active)
            def _(bidx_hbm=bidx_hbm, btab=btab, bout=bout):
                lane = lax.rem(wid, half)
                pltpu.sync_copy(
                    bidx_hbm.at[pl.ds(lane * bias_groups, bias_groups)],
                    idx_v.at[pl.ds(0, bias_groups)])
                handles = [
                    pltpu.async_copy(btab.at[idx_v.at[j]],
                                     bias_v.at[pl.ds(j * 128, 128)], sem0)
                    for j in range(bias_groups)
                ]
                for h in handles:
                    h.wait()
                pltpu.sync_copy(bias_v,
                                bout.at[pl.ds(lane * bias_groups * 128,
                                              bias_groups * 128)])


def _sc_gather_one(table, idx, uidx, iidx, ubias, ibias, n_rows, B, D,
                   do_bias):
    n_steps = (n_rows // NW) // CHUNK
    bias_groups = (B // (NW // 2)) // 128
    mesh = plsc.VectorSubcoreMesh(core_axis_name="c", subcore_axis_name="s")
    body = functools.partial(_sc_gather_kernel, n_rows, n_steps, bias_groups,
                             do_bias)
    out_type = [jax.ShapeDtypeStruct((n_rows, D), jnp.float32),
                jax.ShapeDtypeStruct((B,), jnp.float32),
                jax.ShapeDtypeStruct((B,), jnp.float32)]
    f = pl.kernel(
        body,
        out_type=tuple(out_type),
        mesh=mesh,
        compiler_params=pltpu.CompilerParams(use_tc_tiling_on_sc=False),
        scratch_types=[
            pltpu.VMEM(((n_rows // NW) // 128, 128), jnp.int32),
            pltpu.VMEM((CHUNK, D), jnp.float32),
            pltpu.VMEM((CHUNK, D), jnp.float32),
            pltpu.VMEM((bias_groups * 128,), jnp.float32),
            pltpu.SemaphoreType.DMA,
            pltpu.SemaphoreType.DMA,
        ],
        name="sc_gather_table" + ("_and_biases" if do_bias else ""),
    )
    return f(table, idx, uidx, iidx, ubias, ibias)


def _dot(a, b):
    return jnp.dot(a, b, precision=jax.lax.Precision.DEFAULT,
                   preferred_element_type=jnp.float32)


def _tc_pool_kernel(r_ref_rows, m_ref, wm_ref, e_ref, rm_ref, ab_ref, o_ref):
    absum = jnp.sum(ab_ref[...])
    rows = r_ref_rows[...]
    mask = m_ref[...] != 0
    s = _dot(rows, wm_ref[...]) + absum
    s = jnp.where(mask, s, NEG_INF)
    m = jnp.max(s, axis=-1, keepdims=True)
    e = jnp.exp(s - m)
    d = jnp.sum(e, axis=-1, keepdims=True)
    p = e / d
    pexp = _dot(p, e_ref[...])
    o_ref[0] = _dot(pexp * rows, rm_ref[...])


def _tc_pool_one(rows, idx, wmat, emat, rmat, ab, B, L, D, BE):
    nblk = B // BE
    return pl.pallas_call(
        _tc_pool_kernel,
        grid=(nblk,),
        in_specs=[
            pl.BlockSpec((BE, L * D), lambda i: (i, 0)),
            pl.BlockSpec((BE, L), lambda i: (i, 0)),
            pl.BlockSpec((L * D, L), lambda i: (0, 0)),
            pl.BlockSpec((L, L * D), lambda i: (0, 0)),
            pl.BlockSpec((L * D, D), lambda i: (0, 0)),
            pl.BlockSpec((1, D), lambda i: (0, 0)),
        ],
        out_specs=pl.BlockSpec((1, BE, D), lambda i: (i, 0, 0)),
        out_shape=jax.ShapeDtypeStruct((nblk, BE, D), jnp.float32),
    )(rows.reshape(B, L * D), idx, wmat, emat, rmat, ab.reshape(1, D))


def _tc_combine_kernel(pu_ref, pi_ref, ub_ref, ib_ref, gb_ref, o_ref):
    dot = jnp.sum(pu_ref[0] * pi_ref[0], axis=-1, keepdims=True)
    o_ref[0] = dot + ub_ref[0] + ib_ref[0] + gb_ref[0, 0]


def _tc_combine(pu, pi, ub, ib, gb, B, D, BE):
    nblk = B // BE
    out = pl.pallas_call(
        _tc_combine_kernel,
        grid=(nblk,),
        in_specs=[
            pl.BlockSpec((1, BE, D), lambda i: (i, 0, 0)),
            pl.BlockSpec((1, BE, D), lambda i: (i, 0, 0)),
            pl.BlockSpec((1, BE, 1), lambda i: (i, 0, 0)),
            pl.BlockSpec((1, BE, 1), lambda i: (i, 0, 0)),
            pl.BlockSpec((1, 1), lambda i: (0, 0)),
        ],
        out_specs=pl.BlockSpec((1, BE, 1), lambda i: (i, 0, 0)),
        out_shape=jax.ShapeDtypeStruct((nblk, BE, 1), jnp.float32),
    )(pu, pi, ub.reshape(nblk, BE, 1), ib.reshape(nblk, BE, 1),
      gb.reshape(1, 1))
    return out.reshape(B)


def kernel(user_idx, item_idx, fav_subjects, book_subjects, subj_emb,
           attn_weight, attn_bias, user_bias, item_bias, global_bias):
    B, L = fav_subjects.shape
    D = subj_emb.shape[1]
    n_rows = B * L
    BE = 256

    fidx = fav_subjects.astype(jnp.int32).reshape(n_rows // 128, 128)
    bidx = book_subjects.astype(jnp.int32).reshape(n_rows // 128, 128)
    uidx = user_idx.astype(jnp.int32).reshape(B // 128, 128)
    iidx = item_idx.astype(jnp.int32).reshape(B // 128, 128)
    ubias = user_bias.reshape(-1)
    ibias = item_bias.reshape(-1)

    wmat = jnp.kron(jnp.eye(L, dtype=jnp.float32), attn_weight.reshape(D, 1))
    emat = jnp.kron(jnp.eye(L, dtype=jnp.float32),
                    jnp.ones((1, D), jnp.float32))
    rmat = jnp.kron(jnp.ones((L, 1), jnp.float32),
                    jnp.eye(D, dtype=jnp.float32))

    rows_f, ub, ib = _sc_gather_one(subj_emb, fidx, uidx, iidx, ubias, ibias,
                                    n_rows, B, D, True)
    rows_b, _, _ = _sc_gather_one(subj_emb, bidx, uidx, iidx, ubias, ibias,
                                  n_rows, B, D, False)

    pu = _tc_pool_one(rows_f, fav_subjects.astype(jnp.int32), wmat, emat,
                      rmat, attn_bias, B, L, D, BE)
    pi = _tc_pool_one(rows_b, book_subjects.astype(jnp.int32), wmat, emat,
                      rmat, attn_bias, B, L, D, BE)
    return _tc_combine(pu, pi, ub, ib, global_bias, B, D, BE)
